# Initial kernel scaffold; baseline (speedup 1.0000x reference)
import jax
import jax.numpy as jnp
from jax.experimental import pallas as pl


def kernel(vertices, normals, bone_weights, bone_indices, bone_matrices):
    raise NotImplementedError("write your pallas kernel here")


# speedup vs baseline: 5.3797x; 5.3797x over previous
import functools

import jax
import jax.numpy as jnp
from jax import lax
from jax.experimental import pallas as pl
from jax.experimental.pallas import tpu as pltpu, tpu_sc as plsc

_NW = 32
_CB = 2000
_L = 16


def _build(n, m):
    num_chunks = n // _CB
    groups = _CB // _L
    mesh = plsc.VectorSubcoreMesh(core_axis_name="c", subcore_axis_name="s")

    @functools.partial(
        pl.kernel,
        out_type=(
            jax.ShapeDtypeStruct((n * 3,), jnp.float32),
            jax.ShapeDtypeStruct((n * 3,), jnp.float32),
        ),
        mesh=mesh,
        scratch_types=[
            pltpu.VMEM((m * 16,), jnp.float32),
            pltpu.VMEM((_CB * 3,), jnp.float32),
            pltpu.VMEM((_CB * 3,), jnp.float32),
            pltpu.VMEM((_CB * 4,), jnp.float32),
            pltpu.VMEM((_CB * 4,), jnp.int32),
            pltpu.VMEM((_CB * 3,), jnp.float32),
            pltpu.VMEM((_CB * 3,), jnp.float32),
        ],
        compiler_params=pltpu.CompilerParams(
            use_tc_tiling_on_sc=False, needs_layout_passes=False),
    )
    def skin(v_hbm, nrm_hbm, w_hbm, idx_hbm, tab_hbm,
             ov_hbm, on_hbm,
             tab_v, v_v, n_v, w_v, i_v, ov_v, on_v):
        cid = lax.axis_index("c")
        sid = lax.axis_index("s")
        wid = sid * 2 + cid

        pltpu.sync_copy(tab_hbm, tab_v)

        lane = lax.iota(jnp.int32, _L)
        zero = jnp.zeros((_L,), jnp.float32)

        def group(g, carry):
            rows = lane + g * _L
            f3 = rows * 3
            f4 = rows * 4
            x = plsc.load_gather(v_v, [f3])
            y = plsc.load_gather(v_v, [f3 + 1])
            z = plsc.load_gather(v_v, [f3 + 2])
            nx = plsc.load_gather(n_v, [f3])
            ny = plsc.load_gather(n_v, [f3 + 1])
            nz = plsc.load_gather(n_v, [f3 + 2])
            av0 = av1 = av2 = zero
            an0 = an1 = an2 = zero
            for i in range(4):
                bi = plsc.load_gather(i_v, [f4 + i])
                w = plsc.load_gather(w_v, [f4 + i])
                mb = bi * 16
                mm = [plsc.load_gather(tab_v, [mb + k]) for k in range(16)]
                t0 = x * mm[0] + y * mm[1] + z * mm[2] + mm[3]
                t1 = x * mm[4] + y * mm[5] + z * mm[6] + mm[7]
                t2 = x * mm[8] + y * mm[9] + z * mm[10] + mm[11]
                t3 = x * mm[12] + y * mm[13] + z * mm[14] + mm[15]
                r = w / t3
                av0 = av0 + t0 * r
                av1 = av1 + t1 * r
                av2 = av2 + t2 * r
                an0 = an0 + w * (nx * mm[0] + ny * mm[1] + nz * mm[2])
                an1 = an1 + w * (nx * mm[4] + ny * mm[5] + nz * mm[6])
                an2 = an2 + w * (nx * mm[8] + ny * mm[9] + nz * mm[10])
            plsc.store_scatter(ov_v, [f3], av0)
            plsc.store_scatter(ov_v, [f3 + 1], av1)
            plsc.store_scatter(ov_v, [f3 + 2], av2)
            plsc.store_scatter(on_v, [f3], an0)
            plsc.store_scatter(on_v, [f3 + 1], an1)
            plsc.store_scatter(on_v, [f3 + 2], an2)
            return carry

        def chunk(ci, carry):
            c = wid + ci * _NW
            b3 = c * (_CB * 3)
            b4 = c * (_CB * 4)
            pltpu.sync_copy(v_hbm.at[pl.ds(b3, _CB * 3)], v_v)
            pltpu.sync_copy(nrm_hbm.at[pl.ds(b3, _CB * 3)], n_v)
            pltpu.sync_copy(w_hbm.at[pl.ds(b4, _CB * 4)], w_v)
            pltpu.sync_copy(idx_hbm.at[pl.ds(b4, _CB * 4)], i_v)
            lax.fori_loop(0, groups, group, 0, unroll=False)
            pltpu.sync_copy(ov_v, ov_hbm.at[pl.ds(b3, _CB * 3)])
            pltpu.sync_copy(on_v, on_hbm.at[pl.ds(b3, _CB * 3)])
            return carry

        my_chunks = (num_chunks - 1 - wid) // _NW + 1
        lax.fori_loop(0, my_chunks, chunk, 0, unroll=False)

    return skin


def kernel(vertices, normals, bone_weights, bone_indices, bone_matrices):
    n = vertices.shape[0]
    m = bone_matrices.shape[0]
    pad = (-n) % _CB
    if pad:
        vertices = jnp.pad(vertices, ((0, pad), (0, 0)))
        normals = jnp.pad(normals, ((0, pad), (0, 0)))
        bone_weights = jnp.pad(bone_weights, ((0, pad), (0, 0)))
        bone_indices = jnp.pad(bone_indices, ((0, pad), (0, 0)))
    npad = n + pad
    vf = vertices.reshape(-1).astype(jnp.float32)
    nf = normals.reshape(-1).astype(jnp.float32)
    wf = bone_weights.reshape(-1).astype(jnp.float32)
    idxf = bone_indices.astype(jnp.int32).reshape(-1)
    tf = bone_matrices.astype(jnp.float32).reshape(-1)
    ov, on = _build(npad, m)(vf, nf, wf, idxf, tf)
    return ov.reshape(npad, 3)[:n], on.reshape(npad, 3)[:n]
